# baseline (speedup 1.0000x reference)
import functools

import jax
import jax.numpy as jnp
from jax import lax
from jax.experimental import pallas as pl
from jax.experimental.pallas import tpu as pltpu
from jax.experimental.pallas import tpu_sc as plsc

_NC = 2
_NS = 16
_NW = _NC * _NS
_L = 16


def _rsqrt(a):
    i = lax.bitcast_convert_type(a, jnp.int32)
    y = lax.bitcast_convert_type(
        jnp.int32(0x5F3759DF) - lax.shift_right_logical(i, 1), jnp.float32
    )
    for _ in range(3):
        y = y * (1.5 - 0.5 * a * y * y)
    return y


def _sc_fused(tokens, W, P, gamma, beta):
    B, S = tokens.shape
    V, D = W.shape
    NJ = D // _L
    SPW = S // _NW
    CH = 8
    NSC = SPW // CH
    SCALE = float(D) ** 0.5
    INV_D = 1.0 / D

    mesh = plsc.VectorSubcoreMesh(core_axis_name="c", subcore_axis_name="s")

    @functools.partial(
        pl.kernel,
        mesh=mesh,
        out_type=jax.ShapeDtypeStruct((B, S, D), jnp.float32),
        scratch_types=[
            pltpu.VMEM((B * SPW,), jnp.int32),
            pltpu.VMEM((D,), jnp.float32),
            pltpu.VMEM((D,), jnp.float32),
            pltpu.VMEM((CH, D), jnp.float32),
            pltpu.VMEM((CH, D), jnp.float32),
            pltpu.VMEM((B, CH, D), jnp.float32),
            pltpu.VMEM((B, CH, D), jnp.float32),
            pltpu.SemaphoreType.DMA,
            pltpu.SemaphoreType.DMA,
            pltpu.SemaphoreType.DMA,
            pltpu.SemaphoreType.DMA,
            pltpu.SemaphoreType.DMA,
            pltpu.SemaphoreType.DMA,
        ],
        compiler_params=pltpu.CompilerParams(needs_layout_passes=False),
    )
    def k(tok_hbm, W_hbm, P_hbm, gamma_hbm, beta_hbm, out_hbm,
          idx_v, gam_v, bet_v, pb0, pb1, rb0, rb1,
          gs0, gs1, ps0, ps1, ss0, ss1):
        wid = lax.axis_index("s") * _NC + lax.axis_index("c")
        s0 = wid * SPW
        pbufs = (pb0, pb1)
        rbufs = (rb0, rb1)
        gsems = (gs0, gs1)
        psems = (ps0, ps1)
        ssems = (ss0, ss1)

        for b in range(B):
            pltpu.sync_copy(tok_hbm.at[b, pl.ds(s0, SPW)],
                            idx_v.at[pl.ds(b * SPW, SPW)])
        pltpu.sync_copy(gamma_hbm, gam_v)
        pltpu.sync_copy(beta_hbm, bet_v)

        def p_copy(sc, par):
            return pltpu.make_async_copy(
                P_hbm.at[pl.ds(s0 + sc * CH, CH)], pbufs[par], psems[par])

        def gather_copy(sc, par, b):
            return pltpu.make_async_copy(
                W_hbm.at[idx_v.at[pl.ds(b * SPW + sc * CH, CH)]],
                rbufs[par].at[b], gsems[par])

        def scatter_copy(sc, par, b):
            return pltpu.make_async_copy(
                rbufs[par].at[b],
                out_hbm.at[b, pl.ds(s0 + sc * CH, CH)], ssems[par])

        def fire_group(sc, par):
            p_copy(sc, par).start()
            for b in range(B):
                gather_copy(sc, par, b).start()

        def wait_group(sc, par):
            p_copy(sc, par).wait()
            for b in range(B):
                gather_copy(sc, par, b).wait()

        def compute_group(par):
            rbuf = rbufs[par]
            pbuf = pbufs[par]

            @plsc.parallel_loop(0, CH, unroll=2)
            def rbody(r):
                acc_s = [jnp.zeros((_L,), jnp.float32) for _ in range(B)]
                acc_q = [jnp.zeros((_L,), jnp.float32) for _ in range(B)]
                for j in range(NJ):
                    ds = pl.ds(j * _L, _L)
                    pj = pbuf[r, ds]
                    for b in range(B):
                        x = rbuf[b, r, ds] * SCALE + pj
                        rbuf[b, r, ds] = x
                        acc_s[b] = acc_s[b] + x
                        acc_q[b] = acc_q[b] + x * x
                cs = []
                invs = []
                for b in range(B):
                    mu = jnp.sum(acc_s[b]) * INV_D
                    var = jnp.sum(acc_q[b]) * INV_D - mu * mu
                    inv = _rsqrt(var + 1e-12)
                    invs.append(inv)
                    cs.append(-mu * inv)
                for j in range(NJ):
                    ds = pl.ds(j * _L, _L)
                    gj = gam_v[ds]
                    bj = bet_v[ds]
                    for b in range(B):
                        xn = rbuf[b, r, ds] * invs[b] + cs[b]
                        rbuf[b, r, ds] = xn * gj + bj

        fire_group(0, 0)

        def outer(i, carry):
            sc0 = 2 * i
            sc1 = 2 * i + 1

            @pl.when(i >= 1)
            def _():
                for b in range(B):
                    scatter_copy(sc0 - 1, 1, b).wait()
            fire_group(sc1, 1)
            wait_group(sc0, 0)
            compute_group(0)
            for b in range(B):
                scatter_copy(sc0, 0, b).start()

            @pl.when(i + 1 < NSC // 2)
            def _():
                for b in range(B):
                    scatter_copy(sc0, 0, b).wait()
                fire_group(sc0 + 2, 0)
            wait_group(sc1, 1)
            compute_group(1)
            for b in range(B):
                scatter_copy(sc1, 1, b).start()
            return carry

        lax.fori_loop(0, NSC // 2, outer, 0)
        for b in range(B):
            scatter_copy(NSC - 2, 0, b).wait()
            scatter_copy(NSC - 1, 1, b).wait()

    return k(tokens, W, P, gamma, beta)


def kernel(tokens, att_mask, W, P, gamma, beta):
    out = _sc_fused(tokens, W, P, gamma, beta)
    return out, att_mask


# speedup vs baseline: 1.7629x; 1.7629x over previous
import functools

import jax
import jax.numpy as jnp
from jax import lax
from jax.experimental import pallas as pl
from jax.experimental.pallas import tpu as pltpu
from jax.experimental.pallas import tpu_sc as plsc

_NC = 2
_NS = 16
_NW = _NC * _NS
_L = 16


def _rsqrt(a):
    i = lax.bitcast_convert_type(a, jnp.int32)
    y = lax.bitcast_convert_type(
        jnp.int32(0x5F3759DF) - lax.shift_right_logical(i, 1), jnp.float32
    )
    for _ in range(3):
        y = y * (1.5 - 0.5 * a * y * y)
    return y


def _sc_fused(tokens, W, P, gamma, beta):
    B, S = tokens.shape
    V, D = W.shape
    NJ = D // _L
    SPW = S // _NW
    CH = 8
    NSC = SPW // CH
    SCALE = float(D) ** 0.5
    INV_D = 1.0 / D

    mesh = plsc.VectorSubcoreMesh(core_axis_name="c", subcore_axis_name="s")

    @functools.partial(
        pl.kernel,
        mesh=mesh,
        out_type=jax.ShapeDtypeStruct((B, S, D), jnp.float32),
        scratch_types=[
            pltpu.VMEM((B * SPW,), jnp.int32),
            pltpu.VMEM((D,), jnp.float32),
            pltpu.VMEM((D,), jnp.float32),
            pltpu.VMEM((CH, D), jnp.float32),
            pltpu.VMEM((CH, D), jnp.float32),
            pltpu.VMEM((B, CH, D), jnp.float32),
            pltpu.VMEM((B, CH, D), jnp.float32),
            pltpu.VMEM((B, CH, D), jnp.float32),
            pltpu.SemaphoreType.DMA,
            pltpu.SemaphoreType.DMA,
            pltpu.SemaphoreType.DMA,
            pltpu.SemaphoreType.DMA,
            pltpu.SemaphoreType.DMA,
            pltpu.SemaphoreType.DMA,
        ],
        compiler_params=pltpu.CompilerParams(needs_layout_passes=False),
    )
    def k(tok_hbm, W_hbm, P_hbm, gamma_hbm, beta_hbm, out_hbm,
          idx_v, gam_v, bet_v, pb0, pb1, rb0, rb1, xbuf,
          gs0, gs1, ps0, ps1, ss0, ss1):
        wid = lax.axis_index("s") * _NC + lax.axis_index("c")
        s0 = wid * SPW
        pbufs = (pb0, pb1)
        rbufs = (rb0, rb1)
        gsems = (gs0, gs1)
        psems = (ps0, ps1)
        ssems = (ss0, ss1)

        for b in range(B):
            pltpu.sync_copy(tok_hbm.at[b, pl.ds(s0, SPW)],
                            idx_v.at[pl.ds(b * SPW, SPW)])
        pltpu.sync_copy(gamma_hbm, gam_v)
        pltpu.sync_copy(beta_hbm, bet_v)

        def p_copy(sc, par):
            return pltpu.make_async_copy(
                P_hbm.at[pl.ds(s0 + sc * CH, CH)], pbufs[par], psems[par])

        def gather_copy(sc, par, b):
            return pltpu.make_async_copy(
                W_hbm.at[idx_v.at[pl.ds(b * SPW + sc * CH, CH)]],
                rbufs[par].at[b], gsems[par])

        def scatter_copy(sc, par, b):
            return pltpu.make_async_copy(
                rbufs[par].at[b],
                out_hbm.at[b, pl.ds(s0 + sc * CH, CH)], ssems[par])

        def fire_group(sc, par):
            p_copy(sc, par).start()
            for b in range(B):
                gather_copy(sc, par, b).start()

        def wait_group(sc, par):
            p_copy(sc, par).wait()
            for b in range(B):
                gather_copy(sc, par, b).wait()

        def compute_group(par):
            rbuf = rbufs[par]
            pbuf = pbufs[par]

            @plsc.parallel_loop(0, CH, unroll=1)
            def rbody(r):
                acc_s = [jnp.zeros((_L,), jnp.float32) for _ in range(B)]
                acc_q = [jnp.zeros((_L,), jnp.float32) for _ in range(B)]
                for j in range(NJ):
                    ds = pl.ds(j * _L, _L)
                    pj = pbuf[r, ds]
                    for b in range(B):
                        x = rbuf[b, r, ds] * SCALE + pj
                        xbuf[b, r, ds] = x
                        acc_s[b] = acc_s[b] + x
                        acc_q[b] = acc_q[b] + x * x
                cs = []
                invs = []
                for b in range(B):
                    mu = jnp.sum(acc_s[b]) * INV_D
                    var = jnp.sum(acc_q[b]) * INV_D - mu * mu
                    inv = _rsqrt(var + 1e-12)
                    invs.append(inv)
                    cs.append(-mu * inv)
                for j in range(NJ):
                    ds = pl.ds(j * _L, _L)
                    gj = gam_v[ds]
                    bj = bet_v[ds]
                    for b in range(B):
                        xn = xbuf[b, r, ds] * invs[b] + cs[b]
                        rbuf[b, r, ds] = xn * gj + bj

        fire_group(0, 0)

        def outer(i, carry):
            sc0 = 2 * i
            sc1 = 2 * i + 1

            @pl.when(i >= 1)
            def _():
                for b in range(B):
                    scatter_copy(sc0 - 1, 1, b).wait()
            fire_group(sc1, 1)
            wait_group(sc0, 0)
            compute_group(0)
            for b in range(B):
                scatter_copy(sc0, 0, b).start()

            @pl.when(i + 1 < NSC // 2)
            def _():
                for b in range(B):
                    scatter_copy(sc0, 0, b).wait()
                fire_group(sc0 + 2, 0)
            wait_group(sc1, 1)
            compute_group(1)
            for b in range(B):
                scatter_copy(sc1, 1, b).start()
            return carry

        lax.fori_loop(0, NSC // 2, outer, 0)
        for b in range(B):
            scatter_copy(NSC - 2, 0, b).wait()
            scatter_copy(NSC - 1, 1, b).wait()

    return k(tokens, W, P, gamma, beta)


def kernel(tokens, att_mask, W, P, gamma, beta):
    out = _sc_fused(tokens, W, P, gamma, beta)
    return out, att_mask


# speedup vs baseline: 2.8170x; 1.5979x over previous
import functools

import jax
import jax.numpy as jnp
from jax import lax
from jax.experimental import pallas as pl
from jax.experimental.pallas import tpu as pltpu
from jax.experimental.pallas import tpu_sc as plsc

_NC = 2
_NS = 16
_NW = _NC * _NS


def _sc_gather(tokens_flat, W):
    N = tokens_flat.shape[0]
    V, D = W.shape
    per_w = N // _NW
    CH = 32
    n_ch = per_w // CH

    mesh = plsc.VectorSubcoreMesh(core_axis_name="c", subcore_axis_name="s")

    @functools.partial(
        pl.kernel,
        mesh=mesh,
        out_type=jax.ShapeDtypeStruct((N, D), jnp.float32),
        scratch_types=[
            pltpu.VMEM((per_w,), jnp.int32),
            pltpu.VMEM((CH, D), jnp.float32),
            pltpu.VMEM((CH, D), jnp.float32),
            pltpu.SemaphoreType.DMA,
            pltpu.SemaphoreType.DMA,
            pltpu.SemaphoreType.DMA,
            pltpu.SemaphoreType.DMA,
        ],
    )
    def k(tokens_hbm, W_hbm, out_hbm, idx_v, buf0, buf1, g0, g1, s0, s1):
        wid = lax.axis_index("s") * _NC + lax.axis_index("c")
        base = wid * per_w
        bufs = (buf0, buf1)
        gsems = (g0, g1)
        ssems = (s0, s1)
        pltpu.sync_copy(tokens_hbm.at[pl.ds(base, per_w)], idx_v)

        def gather(c):
            return pltpu.async_copy(
                W_hbm.at[idx_v.at[pl.ds(c * CH, CH)]], bufs[c % 2], gsems[c % 2]
            )

        def scatter(c):
            return pltpu.async_copy(
                bufs[c % 2], out_hbm.at[pl.ds(base + c * CH, CH)], ssems[c % 2]
            )

        pend_g = {0: gather(0)}
        pend_s = {}
        for c in range(n_ch):
            pend_g.pop(c).wait()
            if c + 1 < n_ch:
                if c - 1 in pend_s:
                    pend_s.pop(c - 1).wait()
                pend_g[c + 1] = gather(c + 1)
            pend_s[c] = scatter(c)
        pend_s.pop(n_ch - 1).wait()

    return k(tokens_flat, W)


def _tc_ln(g, P, gamma, beta, d_model):
    N, D = g.shape
    S = P.shape[0]
    B = N // S
    BLK = 512
    scale = float(d_model) ** 0.5

    def body(g_ref, p_ref, gm_ref, bt_ref, o_ref):
        x = g_ref[...] * scale + p_ref[...]
        mu = jnp.mean(x, axis=-1, keepdims=True)
        var = jnp.mean((x - mu) ** 2, axis=-1, keepdims=True)
        xn = (x - mu) / jnp.sqrt(var + 1e-12)
        o_ref[...] = xn * gm_ref[...] + bt_ref[...]

    n_s = S // BLK
    return pl.pallas_call(
        body,
        grid=(n_s, B),
        in_specs=[
            pl.BlockSpec((BLK, D), lambda i, b: (b * n_s + i, 0)),
            pl.BlockSpec((BLK, D), lambda i, b: (i, 0)),
            pl.BlockSpec((1, D), lambda i, b: (0, 0)),
            pl.BlockSpec((1, D), lambda i, b: (0, 0)),
        ],
        out_specs=pl.BlockSpec((BLK, D), lambda i, b: (b * n_s + i, 0)),
        out_shape=jax.ShapeDtypeStruct((N, D), jnp.float32),
    )(g, P, gamma.reshape(1, D), beta.reshape(1, D))


def kernel(tokens, att_mask, W, P, gamma, beta):
    B, S = tokens.shape
    D = W.shape[1]
    g = _sc_gather(tokens.reshape(-1), W)
    out = _tc_ln(g, P, gamma, beta, D)
    return out.reshape(B, S, D), att_mask


# speedup vs baseline: 2.9600x; 1.0508x over previous
import functools

import jax
import jax.numpy as jnp
from jax import lax
from jax.experimental import pallas as pl
from jax.experimental.pallas import tpu as pltpu
from jax.experimental.pallas import tpu_sc as plsc

_NC = 2
_NS = 16
_NW = _NC * _NS


def _sc_gather(tokens_flat, W):
    N = tokens_flat.shape[0]
    V, D = W.shape
    per_w = N // _NW
    CH = 32
    n_ch = per_w // CH

    mesh = plsc.VectorSubcoreMesh(core_axis_name="c", subcore_axis_name="s")

    @functools.partial(
        pl.kernel,
        mesh=mesh,
        out_type=jax.ShapeDtypeStruct((N, D), jnp.float32),
        scratch_types=[
            pltpu.VMEM((per_w,), jnp.int32),
            pltpu.VMEM((CH, D), jnp.float32),
            pltpu.VMEM((CH, D), jnp.float32),
            pltpu.SemaphoreType.DMA,
            pltpu.SemaphoreType.DMA,
            pltpu.SemaphoreType.DMA,
            pltpu.SemaphoreType.DMA,
        ],
    )
    def k(tokens_hbm, W_hbm, out_hbm, idx_v, buf0, buf1, g0, g1, s0, s1):
        wid = lax.axis_index("s") * _NC + lax.axis_index("c")
        base = wid * per_w
        bufs = (buf0, buf1)
        gsems = (g0, g1)
        ssems = (s0, s1)
        pltpu.sync_copy(tokens_hbm.at[pl.ds(base, per_w)], idx_v)

        def gather(c):
            return pltpu.async_copy(
                W_hbm.at[idx_v.at[pl.ds(c * CH, CH)]], bufs[c % 2], gsems[c % 2]
            )

        def scatter(c):
            return pltpu.async_copy(
                bufs[c % 2], out_hbm.at[pl.ds(base + c * CH, CH)], ssems[c % 2]
            )

        pend_g = {0: gather(0)}
        pend_s = {}
        for c in range(n_ch):
            pend_g.pop(c).wait()
            if c + 1 < n_ch:
                if c - 1 in pend_s:
                    pend_s.pop(c - 1).wait()
                pend_g[c + 1] = gather(c + 1)
            pend_s[c] = scatter(c)
        pend_s.pop(n_ch - 1).wait()

    return k(tokens_flat, W)


def _tc_ln(g, P, gamma, beta, d_model):
    N, D = g.shape
    S = P.shape[0]
    B = N // S
    BLK = 1024
    scale = float(d_model) ** 0.5

    def body(g_ref, p_ref, gm_ref, bt_ref, o_ref):
        x = g_ref[...] * scale + p_ref[...]
        mu = jnp.mean(x, axis=-1, keepdims=True)
        var = jnp.mean((x - mu) ** 2, axis=-1, keepdims=True)
        xn = (x - mu) / jnp.sqrt(var + 1e-12)
        o_ref[...] = xn * gm_ref[...] + bt_ref[...]

    n_s = S // BLK
    return pl.pallas_call(
        body,
        grid=(n_s, B),
        in_specs=[
            pl.BlockSpec((BLK, D), lambda i, b: (b * n_s + i, 0)),
            pl.BlockSpec((BLK, D), lambda i, b: (i, 0)),
            pl.BlockSpec((1, D), lambda i, b: (0, 0)),
            pl.BlockSpec((1, D), lambda i, b: (0, 0)),
        ],
        out_specs=pl.BlockSpec((BLK, D), lambda i, b: (b * n_s + i, 0)),
        out_shape=jax.ShapeDtypeStruct((N, D), jnp.float32),
    )(g, P, gamma.reshape(1, D), beta.reshape(1, D))


def kernel(tokens, att_mask, W, P, gamma, beta):
    B, S = tokens.shape
    D = W.shape[1]
    g = _sc_gather(tokens.reshape(-1), W)
    out = _tc_ln(g, P, gamma, beta, D)
    return out.reshape(B, S, D), att_mask
